# baseline (speedup 1.0000x reference)
import functools

import jax
import jax.numpy as jnp
from jax.experimental import pallas as pl
from jax.experimental.pallas import tpu as pltpu
from jax.experimental.pallas import tpu_sc as plsc

_B = 4096
_K = 8192
_D = 64
_TILE = 128
_GRID = _B // _TILE
_NB = 4
_COMMITMENT_COST = 0.25


def _vq_kernel(x_ref, emb_ref, enc_ref, q_ref, loss_ref, perp_ref, idx_ref,
               esq_ref, counts_ref, loss_acc_ref, zbuf_ref, sems):
    i = pl.program_id(0)

    @pl.when(i == 0)
    def _init():
        emb0 = emb_ref[...]
        esq_ref[...] = jnp.sum(emb0 * emb0, axis=1)[None, :]
        counts_ref[...] = jnp.zeros_like(counts_ref)
        loss_acc_ref[0, 0] = 0.0
        zbuf_ref[...] = jnp.zeros_like(zbuf_ref)

    slot = jax.lax.rem(i, _NB)
    for b in range(_NB):
        @pl.when(slot == b)
        def _send(b=b):
            @pl.when(i >= _NB)
            def _reclaim():
                pltpu.make_async_copy(
                    zbuf_ref, enc_ref.at[pl.ds((i - _NB) * _TILE, _TILE)],
                    sems.at[b]).wait()
            pltpu.make_async_copy(
                zbuf_ref, enc_ref.at[pl.ds(i * _TILE, _TILE)],
                sems.at[b]).start()

    x = x_ref[...]
    emb = emb_ref[...]
    xsq = jnp.sum(x * x, axis=1, keepdims=True)
    prod2 = jax.lax.dot_general(x + x, emb, (((1,), (1,)), ((), ())),
                                preferred_element_type=jnp.float32)
    dist = (xsq + esq_ref[...]) - prod2
    idx = jnp.argmin(dist, axis=1)
    idx_ref[...] = idx.reshape(1, 1, _TILE)
    onehot = (jax.lax.broadcasted_iota(jnp.int32, (_TILE, _K), 1)
              == idx[:, None]).astype(jnp.float32)
    q = jax.lax.dot_general(onehot, emb, (((1,), (0,)), ((), ())),
                            preferred_element_type=jnp.float32)
    q_ref[...] = x + (q - x)
    counts_ref[...] += jax.lax.dot_general(
        jnp.ones((1, _TILE), jnp.float32), onehot, (((1,), (0,)), ((), ())),
        preferred_element_type=jnp.float32)
    diff = q - x
    loss_acc_ref[0, 0] += jnp.sum(diff * diff)

    @pl.when(i == _GRID - 1)
    def _fin():
        for b in range(_NB):
            pltpu.make_async_copy(
                zbuf_ref, enc_ref.at[pl.ds(0, _TILE)], sems.at[b]).wait()
        m = loss_acc_ref[0, 0] / (_B * _D)
        loss_ref[0, 0] = m + _COMMITMENT_COST * m
        probs = counts_ref[...] * (1.0 / _B)
        ent = -jnp.sum(probs * jnp.log(probs + 1e-10))
        perp_ref[0, 0] = jnp.exp(ent)


def _tc_call(flat, emb):
    return pl.pallas_call(
        _vq_kernel,
        grid=(_GRID,),
        in_specs=[
            pl.BlockSpec((_TILE, _D), lambda i: (i, 0)),
            pl.BlockSpec((_K, _D), lambda i: (0, 0)),
        ],
        out_specs=[
            pl.BlockSpec(memory_space=pl.ANY),
            pl.BlockSpec((_TILE, _D), lambda i: (i, 0)),
            pl.BlockSpec(memory_space=pltpu.SMEM),
            pl.BlockSpec(memory_space=pltpu.SMEM),
            pl.BlockSpec((1, 1, _TILE), lambda i: (i, 0, 0)),
        ],
        out_shape=[
            jax.ShapeDtypeStruct((_B, _K), jnp.float32),
            jax.ShapeDtypeStruct((_B, _D), jnp.float32),
            jax.ShapeDtypeStruct((1, 1), jnp.float32),
            jax.ShapeDtypeStruct((1, 1), jnp.float32),
            jax.ShapeDtypeStruct((_GRID, 1, _TILE), jnp.int32),
        ],
        scratch_shapes=[
            pltpu.VMEM((1, _K), jnp.float32),
            pltpu.VMEM((1, _K), jnp.float32),
            pltpu.SMEM((1, 1), jnp.float32),
            pltpu.VMEM((_TILE, _K), jnp.float32),
            pltpu.SemaphoreType.DMA((_NB,)),
        ],
    )(flat, emb)


_CW = 128


def _make_sc_ones():
    info = plsc.get_sparse_core_info()
    nc, ns, nl = info.num_cores, info.num_subcores, info.num_lanes
    nw = nc * ns
    rows_w = _B // nw
    chunks_row = _K // _CW
    mesh = plsc.VectorSubcoreMesh(core_axis_name="c", subcore_axis_name="s")

    @functools.partial(
        pl.kernel,
        out_type=jax.ShapeDtypeStruct((_CW,), jnp.float32),
        mesh=mesh,
        compiler_params=pltpu.CompilerParams(has_side_effects=True,
                                             needs_layout_passes=False),
        scratch_types=[
            pltpu.VMEM((rows_w,), jnp.int32),
            pltpu.VMEM((rows_w,), jnp.int32),
            pltpu.VMEM((rows_w, _CW), jnp.float32),
        ],
    )
    def sc_ones(idx_hbm, enc128_hbm, tok_ref, idx_v, cid_v, patch_v):
        wid = jax.lax.axis_index("s") * nc + jax.lax.axis_index("c")
        base = wid * rows_w
        pltpu.sync_copy(idx_hbm.at[pl.ds(base, rows_w)], idx_v)
        iota = jax.lax.broadcasted_iota(jnp.int32, (nl,), 0)
        ones = jnp.ones((nl,), jnp.float32)
        zeros16 = jnp.zeros((nl,), jnp.float32)
        for r in range(rows_w):
            for s in range(_CW // nl):
                patch_v[r, pl.ds(s * nl, nl)] = zeros16
        for j in range(rows_w // nl):
            idx16 = idx_v[pl.ds(j * nl, nl)]
            rows16 = j * nl + iota
            cid_v[pl.ds(j * nl, nl)] = (
                (base + rows16) * chunks_row + jnp.right_shift(idx16, 7))
            plsc.store_scatter(patch_v, [rows16, jnp.bitwise_and(idx16, 127)],
                               ones)
        pltpu.sync_copy(patch_v, enc128_hbm.at[cid_v])
        @pl.when(wid == 0)
        def _tok():
            pltpu.sync_copy(patch_v.at[0], tok_ref)

    return sc_ones


def kernel(inputs, object_classes, embeddings):
    b = inputs.shape[0]
    flat = inputs.reshape(b, -1)
    enc0, q, loss, perp, idx3 = _tc_call(flat, embeddings)
    enc128 = enc0.reshape(_B * _K // _CW, _CW)
    tok = _make_sc_ones()(idx3.reshape(_B), enc128)
    enc128b, _ = jax.lax.optimization_barrier((enc128, tok))
    return (loss[0, 0], q.reshape(inputs.shape), perp[0, 0],
            enc128b.reshape(_B, _K), object_classes)


# speedup vs baseline: 1.4911x; 1.4911x over previous
import functools

import jax
import jax.numpy as jnp
from jax.experimental import pallas as pl
from jax.experimental.pallas import tpu as pltpu
from jax.experimental.pallas import tpu_sc as plsc

_B = 4096
_K = 8192
_D = 64
_TILE = 128
_GRID = _B // _TILE
_NB = 4
_COMMITMENT_COST = 0.25


def _vq_kernel(x_ref, emb_ref, enc_ref, q_ref, loss_ref, perp_ref, idx_ref,
               esq_ref, counts_ref, loss_acc_ref, zbuf_ref, sems):
    i = pl.program_id(0)

    @pl.when(i == 0)
    def _init():
        emb0 = emb_ref[...]
        esq_ref[...] = jnp.sum(emb0 * emb0, axis=1)[None, :]
        counts_ref[...] = jnp.zeros_like(counts_ref)
        loss_acc_ref[0, 0] = 0.0
        zbuf_ref[...] = jnp.zeros_like(zbuf_ref)

    slot = jax.lax.rem(i, _NB)
    for b in range(_NB):
        @pl.when(slot == b)
        def _send(b=b):
            @pl.when(i >= _NB)
            def _reclaim():
                pltpu.make_async_copy(
                    zbuf_ref, enc_ref.at[pl.ds((i - _NB) * _TILE, _TILE)],
                    sems.at[b]).wait()
            pltpu.make_async_copy(
                zbuf_ref, enc_ref.at[pl.ds(i * _TILE, _TILE)],
                sems.at[b]).start()

    x = x_ref[...]
    emb = emb_ref[...]
    xsq = jnp.sum(x * x, axis=1, keepdims=True)
    prod2 = jax.lax.dot_general(x + x, emb, (((1,), (1,)), ((), ())),
                                preferred_element_type=jnp.float32)
    dist = (xsq + esq_ref[...]) - prod2
    idx = jnp.argmin(dist, axis=1)
    idx_ref[...] = idx.reshape(1, 1, _TILE)
    onehot = (jax.lax.broadcasted_iota(jnp.int32, (_TILE, _K), 1)
              == idx[:, None]).astype(jnp.float32)
    q = jax.lax.dot_general(onehot, emb, (((1,), (0,)), ((), ())),
                            preferred_element_type=jnp.float32)
    q_ref[...] = x + (q - x)
    counts_ref[...] += jax.lax.dot_general(
        jnp.ones((1, _TILE), jnp.float32), onehot, (((1,), (0,)), ((), ())),
        preferred_element_type=jnp.float32)
    diff = q - x
    loss_acc_ref[0, 0] += jnp.sum(diff * diff)

    @pl.when(i == _GRID - 1)
    def _fin():
        for b in range(_NB):
            pltpu.make_async_copy(
                zbuf_ref, enc_ref.at[pl.ds(0, _TILE)], sems.at[b]).wait()
        m = loss_acc_ref[0, 0] / (_B * _D)
        loss_ref[0, 0] = m + _COMMITMENT_COST * m
        probs = counts_ref[...] * (1.0 / _B)
        ent = -jnp.sum(probs * jnp.log(probs + 1e-10))
        perp_ref[0, 0] = jnp.exp(ent)


def _tc_call(flat, emb):
    return pl.pallas_call(
        _vq_kernel,
        grid=(_GRID,),
        in_specs=[
            pl.BlockSpec((_TILE, _D), lambda i: (i, 0)),
            pl.BlockSpec((_K, _D), lambda i: (0, 0)),
        ],
        out_specs=[
            pl.BlockSpec(memory_space=pl.ANY),
            pl.BlockSpec((_TILE, _D), lambda i: (i, 0)),
            pl.BlockSpec(memory_space=pltpu.SMEM),
            pl.BlockSpec(memory_space=pltpu.SMEM),
            pl.BlockSpec((1, 1, _TILE), lambda i: (i, 0, 0)),
        ],
        out_shape=[
            jax.ShapeDtypeStruct((_B, _K), jnp.float32),
            jax.ShapeDtypeStruct((_B, _D), jnp.float32),
            jax.ShapeDtypeStruct((1, 1), jnp.float32),
            jax.ShapeDtypeStruct((1, 1), jnp.float32),
            jax.ShapeDtypeStruct((_GRID, 1, _TILE), jnp.int32),
        ],
        scratch_shapes=[
            pltpu.VMEM((1, _K), jnp.float32),
            pltpu.VMEM((1, _K), jnp.float32),
            pltpu.SMEM((1, 1), jnp.float32),
            pltpu.VMEM((_TILE, _K), jnp.float32),
            pltpu.SemaphoreType.DMA((_NB,)),
        ],
    )(flat, emb)


_CW = 128


def _make_sc_ones():
    info = plsc.get_sparse_core_info()
    nc, ns, nl = info.num_cores, info.num_subcores, info.num_lanes
    nw = nc * ns
    rows_w = _B // nw
    chunks_row = _K // _CW
    mesh = plsc.VectorSubcoreMesh(core_axis_name="c", subcore_axis_name="s")

    @functools.partial(
        pl.kernel,
        out_type=jax.ShapeDtypeStruct((_CW,), jnp.float32),
        mesh=mesh,
        compiler_params=pltpu.CompilerParams(has_side_effects=True,
                                             needs_layout_passes=False),
        scratch_types=[
            pltpu.VMEM((rows_w,), jnp.int32),
            pltpu.VMEM((rows_w,), jnp.int32),
            pltpu.VMEM((rows_w, _CW), jnp.float32),
        ],
    )
    def sc_ones(idx_hbm, enc128_hbm, tok_ref, idx_v, cid_v, patch_v):
        wid = jax.lax.axis_index("s") * nc + jax.lax.axis_index("c")
        base = wid * rows_w
        pltpu.sync_copy(idx_hbm.at[pl.ds(base, rows_w)], idx_v)
        iota = jax.lax.broadcasted_iota(jnp.int32, (nl,), 0)
        ones = jnp.ones((nl,), jnp.float32)
        zeros16 = jnp.zeros((nl,), jnp.float32)
        for r in range(rows_w):
            for s in range(_CW // nl):
                patch_v[r, pl.ds(s * nl, nl)] = zeros16
        for j in range(rows_w // nl):
            idx16 = idx_v[pl.ds(j * nl, nl)]
            rows16 = j * nl + iota
            cid_v[pl.ds(j * nl, nl)] = (
                (base + rows16) * chunks_row + jnp.right_shift(idx16, 7))
            plsc.store_scatter(patch_v, [rows16, jnp.bitwise_and(idx16, 127)],
                               ones)
        pltpu.sync_copy(patch_v, enc128_hbm.at[cid_v])
        @pl.when(wid == 0)
        def _tok():
            pltpu.sync_copy(patch_v.at[0], tok_ref)

    return sc_ones


def kernel(inputs, object_classes, embeddings):
    b = inputs.shape[0]
    flat = inputs.reshape(b, -1)
    enc0, q, loss, perp, idx3 = _tc_call(flat, embeddings)
    enc128 = enc0.reshape(_B * _K // _CW, _CW)
    tok = _make_sc_ones()(idx3.reshape(_B), enc128)
    loss_f = loss[0, 0] + jnp.sum(tok) * 0.0
    return (loss_f, q.reshape(inputs.shape), perp[0, 0],
            enc0, object_classes)


# speedup vs baseline: 3.1310x; 2.0998x over previous
import jax
import jax.numpy as jnp
from jax.experimental import pallas as pl
from jax.experimental.pallas import tpu as pltpu

_B = 4096
_K = 8192
_D = 64
_TILE = 64
_GRID = _B // _TILE
_COMMITMENT_COST = 0.25


def _vq_kernel(x_ref, emb_ref, enc_ref, q_ref, loss_ref, perp_ref,
               esq_ref, counts_ref, loss_acc_ref):
    i = pl.program_id(0)

    @pl.when(i == 0)
    def _init():
        emb0 = emb_ref[...]
        esq_ref[...] = jnp.sum(emb0 * emb0, axis=1)[None, :]
        counts_ref[...] = jnp.zeros_like(counts_ref)
        loss_acc_ref[0, 0] = 0.0

    x = x_ref[...]
    emb = emb_ref[...]
    xsq = jnp.sum(x * x, axis=1, keepdims=True)
    prod2 = jax.lax.dot_general(x + x, emb, (((1,), (1,)), ((), ())),
                                preferred_element_type=jnp.float32)
    dist = (xsq + esq_ref[...]) - prod2
    idx = jnp.argmin(dist, axis=1)
    onehot = (jax.lax.broadcasted_iota(jnp.int32, (_TILE, _K), 1)
              == idx[:, None]).astype(jnp.float32)
    enc_ref[...] = onehot
    q = jax.lax.dot_general(onehot, emb, (((1,), (0,)), ((), ())),
                            preferred_element_type=jnp.float32)
    q_ref[...] = x + (q - x)
    counts_ref[...] += jax.lax.dot_general(
        jnp.ones((1, _TILE), jnp.float32), onehot, (((1,), (0,)), ((), ())),
        preferred_element_type=jnp.float32)
    diff = q - x
    loss_acc_ref[0, 0] += jnp.sum(diff * diff)

    @pl.when(i == _GRID - 1)
    def _fin():
        m = loss_acc_ref[0, 0] / (_B * _D)
        loss_ref[0, 0] = m + _COMMITMENT_COST * m
        probs = counts_ref[...] * (1.0 / _B)
        ent = -jnp.sum(probs * jnp.log(probs + 1e-10))
        perp_ref[0, 0] = jnp.exp(ent)


def kernel(inputs, object_classes, embeddings):
    b = inputs.shape[0]
    flat = inputs.reshape(b, -1)
    enc, q, loss, perp = pl.pallas_call(
        _vq_kernel,
        grid=(_GRID,),
        in_specs=[
            pl.BlockSpec((_TILE, _D), lambda i: (i, 0)),
            pl.BlockSpec((_K, _D), lambda i: (0, 0)),
        ],
        out_specs=[
            pl.BlockSpec((_TILE, _K), lambda i: (i, 0)),
            pl.BlockSpec((_TILE, _D), lambda i: (i, 0)),
            pl.BlockSpec(memory_space=pltpu.SMEM),
            pl.BlockSpec(memory_space=pltpu.SMEM),
        ],
        out_shape=[
            jax.ShapeDtypeStruct((_B, _K), jnp.float32),
            jax.ShapeDtypeStruct((_B, _D), jnp.float32),
            jax.ShapeDtypeStruct((1, 1), jnp.float32),
            jax.ShapeDtypeStruct((1, 1), jnp.float32),
        ],
        scratch_shapes=[
            pltpu.VMEM((1, _K), jnp.float32),
            pltpu.VMEM((1, _K), jnp.float32),
            pltpu.SMEM((1, 1), jnp.float32),
        ],
    )(flat, embeddings)
    return (loss[0, 0], q.reshape(inputs.shape), perp[0, 0], enc,
            object_classes)


# speedup vs baseline: 4.2425x; 1.3550x over previous
import jax
import jax.numpy as jnp
from jax.experimental import pallas as pl
from jax.experimental.pallas import tpu as pltpu

_B = 4096
_K = 8192
_D = 64
_TILE = 128
_GRID = _B // _TILE
_COMMITMENT_COST = 0.25


def _vq_kernel(x_ref, emb_ref, enc_ref, q_ref, loss_ref, perp_ref,
               esq_ref, counts_ref, loss_acc_ref):
    i = pl.program_id(0)

    @pl.when(i == 0)
    def _init():
        emb0 = emb_ref[...]
        esq_ref[...] = jnp.sum(emb0 * emb0, axis=1)[None, :]
        counts_ref[...] = jnp.zeros_like(counts_ref)
        loss_acc_ref[0, 0] = 0.0

    x = x_ref[...]
    emb = emb_ref[...]
    xsq = jnp.sum(x * x, axis=1, keepdims=True)
    prod2 = jax.lax.dot_general(x + x, emb, (((1,), (1,)), ((), ())),
                                preferred_element_type=jnp.float32)
    dist = (xsq + esq_ref[...]) - prod2
    idx = jnp.argmin(dist, axis=1)
    onehot = (jax.lax.broadcasted_iota(jnp.int32, (_TILE, _K), 1)
              == idx[:, None]).astype(jnp.float32)
    enc_ref[...] = onehot
    q = jax.lax.dot_general(onehot, emb, (((1,), (0,)), ((), ())),
                            preferred_element_type=jnp.float32)
    q_ref[...] = x + (q - x)
    counts_ref[...] += jax.lax.dot_general(
        jnp.ones((1, _TILE), jnp.float32), onehot, (((1,), (0,)), ((), ())),
        preferred_element_type=jnp.float32)
    diff = q - x
    loss_acc_ref[0, 0] += jnp.sum(diff * diff)

    @pl.when(i == _GRID - 1)
    def _fin():
        m = loss_acc_ref[0, 0] / (_B * _D)
        loss_ref[0, 0] = m + _COMMITMENT_COST * m
        probs = counts_ref[...] * (1.0 / _B)
        ent = -jnp.sum(probs * jnp.log(probs + 1e-10))
        perp_ref[0, 0] = jnp.exp(ent)


def kernel(inputs, object_classes, embeddings):
    b = inputs.shape[0]
    flat = inputs.reshape(b, -1)
    enc, q, loss, perp = pl.pallas_call(
        _vq_kernel,
        grid=(_GRID,),
        in_specs=[
            pl.BlockSpec((_TILE, _D), lambda i: (i, 0)),
            pl.BlockSpec((_K, _D), lambda i: (0, 0)),
        ],
        out_specs=[
            pl.BlockSpec((_TILE, _K), lambda i: (i, 0)),
            pl.BlockSpec((_TILE, _D), lambda i: (i, 0)),
            pl.BlockSpec(memory_space=pltpu.SMEM),
            pl.BlockSpec(memory_space=pltpu.SMEM),
        ],
        out_shape=[
            jax.ShapeDtypeStruct((_B, _K), jnp.float32),
            jax.ShapeDtypeStruct((_B, _D), jnp.float32),
            jax.ShapeDtypeStruct((1, 1), jnp.float32),
            jax.ShapeDtypeStruct((1, 1), jnp.float32),
        ],
        scratch_shapes=[
            pltpu.VMEM((1, _K), jnp.float32),
            pltpu.VMEM((1, _K), jnp.float32),
            pltpu.SMEM((1, 1), jnp.float32),
        ],
    )(flat, embeddings)
    return (loss[0, 0], q.reshape(inputs.shape), perp[0, 0], enc,
            object_classes)
